# baseline (speedup 1.0000x reference)
import jax
import jax.numpy as jnp
from jax.experimental import pallas as pl
from jax.experimental.pallas import tpu as pltpu

B = 8192
D = 4096
H = 256
PROJ = 16
E = 64
K = 8

BM = 1024
NBLK = B // BM


def _gating_kernel(temp_ref, x_ref, w1_ref, b1_ref, w2_ref, b2_ref, en_ref,
                   out_ref, lt_ref):
    f32 = jnp.float32

    logits_t = lt_ref[...]
    neg = jnp.float32(-jnp.inf)
    cur = logits_t
    mx = None
    for k in range(K):
        m = jnp.max(cur, axis=0, keepdims=True)
        if k == 0:
            mx = m
        cur = jnp.where(cur == m, neg, cur)
    p = jnp.where(cur == neg, jnp.exp(logits_t - mx), 0.0)
    g = p / jnp.sum(p, axis=0, keepdims=True)
    out_ref[...] = g.T

    xb = x_ref[...].astype(jnp.bfloat16)
    h = jnp.dot(xb, w1_ref[...], preferred_element_type=f32)
    h = jnp.maximum(h + b1_ref[...], 0.0)
    proj = jnp.dot(h.astype(jnp.bfloat16), w2_ref[...],
                   preferred_element_type=f32)
    proj = proj + b2_ref[...]
    pn = proj * jax.lax.rsqrt(
        jnp.maximum(jnp.sum(proj * proj, axis=1, keepdims=True), 1e-12))
    pn = pn / temp_ref[0, 0]
    en = en_ref[...]
    en_n = en * jax.lax.rsqrt(
        jnp.maximum(jnp.sum(en * en, axis=1, keepdims=True), 1e-12))
    lt_ref[...] = jax.lax.dot_general(
        en_n.astype(jnp.bfloat16), pn.astype(jnp.bfloat16),
        (((1,), (1,)), ((), ())), preferred_element_type=f32)


@jax.jit
def kernel(x, W1, b1, W2, b2, expert_embedding, temperature):
    w1 = W1.astype(jnp.bfloat16)
    w2 = W2.astype(jnp.bfloat16)
    b1r = b1.reshape(1, H)
    b2r = b2.reshape(1, PROJ)
    temp = temperature.reshape(1, 1)

    const = lambda i: (0, 0)
    x_map = lambda i: (jnp.minimum(i, NBLK - 1), 0)
    out_map = lambda i: (jnp.maximum(i - 1, 0), 0)
    out = pl.pallas_call(
        _gating_kernel,
        grid=(NBLK + 1,),
        in_specs=[
            pl.BlockSpec(memory_space=pltpu.SMEM),
            pl.BlockSpec((BM, D), x_map),
            pl.BlockSpec((D, H), const),
            pl.BlockSpec((1, H), const),
            pl.BlockSpec((H, PROJ), const),
            pl.BlockSpec((1, PROJ), const),
            pl.BlockSpec((E, PROJ), const),
        ],
        out_specs=pl.BlockSpec((BM, E), out_map),
        out_shape=jax.ShapeDtypeStruct((B, E), jnp.float32),
        scratch_shapes=[pltpu.VMEM((E, BM), jnp.float32)],
        compiler_params=pltpu.CompilerParams(
            dimension_semantics=("arbitrary",)),
    )(temp, x, w1, b1r, w2, b2r, expert_embedding)
    return out


# speedup vs baseline: 1.0524x; 1.0524x over previous
import jax
import jax.numpy as jnp
from jax.experimental import pallas as pl
from jax.experimental.pallas import tpu as pltpu

B = 8192
D = 4096
H = 256
PROJ = 16
E = 64
K = 8

BM = 1024


def _gating_kernel(temp_ref, x_ref, w1_ref, b1_ref, w2_ref, b2_ref, en_ref,
                   out_ref):
    f32 = jnp.float32
    xb = x_ref[...].astype(jnp.bfloat16)
    h = jnp.dot(xb, w1_ref[...], preferred_element_type=f32)
    h = jnp.maximum(h + b1_ref[...], 0.0)
    proj = jnp.dot(h.astype(jnp.bfloat16), w2_ref[...],
                   preferred_element_type=f32)
    proj = proj + b2_ref[...]
    pn = proj * jax.lax.rsqrt(
        jnp.maximum(jnp.sum(proj * proj, axis=1, keepdims=True), 1e-12))
    pn = pn / temp_ref[0, 0]
    en = en_ref[...]
    en_n = en * jax.lax.rsqrt(
        jnp.maximum(jnp.sum(en * en, axis=1, keepdims=True), 1e-12))
    logits_t = jax.lax.dot_general(
        en_n.astype(jnp.bfloat16), pn.astype(jnp.bfloat16),
        (((1,), (1,)), ((), ())), preferred_element_type=f32)

    neg = jnp.float32(-jnp.inf)
    cur = logits_t
    mx = None
    for k in range(K):
        m = jnp.max(cur, axis=0, keepdims=True)
        if k == 0:
            mx = m
        cur = jnp.where(cur == m, neg, cur)

    p = jnp.where(cur == neg, jnp.exp(logits_t - mx), 0.0)
    g = p / jnp.sum(p, axis=0, keepdims=True)
    out_ref[...] = g.T


@jax.jit
def kernel(x, W1, b1, W2, b2, expert_embedding, temperature):
    w1 = W1.astype(jnp.bfloat16)
    w2 = W2.astype(jnp.bfloat16)
    b1r = b1.reshape(1, H)
    b2r = b2.reshape(1, PROJ)
    temp = temperature.reshape(1, 1)

    grid = (B // BM,)
    const = lambda i: (0, 0)
    out = pl.pallas_call(
        _gating_kernel,
        grid=grid,
        in_specs=[
            pl.BlockSpec(memory_space=pltpu.SMEM),
            pl.BlockSpec((BM, D), lambda i: (i, 0)),
            pl.BlockSpec((D, H), const),
            pl.BlockSpec((1, H), const),
            pl.BlockSpec((H, PROJ), const),
            pl.BlockSpec((1, PROJ), const),
            pl.BlockSpec((E, PROJ), const),
        ],
        out_specs=pl.BlockSpec((BM, E), lambda i: (i, 0)),
        out_shape=jax.ShapeDtypeStruct((B, E), jnp.float32),
        compiler_params=pltpu.CompilerParams(
            dimension_semantics=("parallel",)),
    )(temp, x, w1, b1r, w2, b2r, expert_embedding)
    return out
